# baseline (speedup 1.0000x reference)
import jax
import jax.numpy as jnp
from jax import lax
from jax.experimental import pallas as pl
from jax.experimental.pallas import tpu as pltpu
from jax.experimental.pallas import tpu_sc as plsc

NUM_USERS = 100000
NUM_ITEMS = 1000
BATCH = 16384
HIST = 200

NC, NS, L = 2, 16, 16
NW = NC * NS
BPW = BATCH // NW
C = 32
NCHUNK = BPW // C
VPC = C * HIST // L


def _body(user_hbm, items_hbm, rowbase_hbm, score_hbm, out_hbm,
          uid_v, items_v, rowbase_v, rows_v, out_v, sem):
    wid = lax.axis_index("s") * NC + lax.axis_index("c")
    pltpu.sync_copy(rowbase_hbm, rowbase_v)

    def chunk(g, carry):
        base = wid * BPW + g * C
        pltpu.sync_copy(user_hbm.at[pl.ds(base, C)], uid_v)
        uvecs = [uid_v[pl.ds(k * L, L)] for k in range(C // L)]
        copies = [
            pltpu.async_copy(score_hbm.at[uvecs[i // L][i % L], :],
                             rows_v.at[i, :], sem)
            for i in range(C)
        ]
        pltpu.sync_copy(items_hbm.at[pl.ds(base * HIST, C * HIST)], items_v)
        for cp in copies:
            cp.wait()

        def step(v, c2):
            sl = pl.ds(v * L, L)
            out_v[sl] = plsc.load_gather(rows_v, [rowbase_v[sl], items_v[sl]])
            return c2

        lax.fori_loop(0, VPC, step, 0)
        pltpu.sync_copy(out_v, out_hbm.at[pl.ds(base * HIST, C * HIST)])
        return carry

    lax.fori_loop(0, NCHUNK, chunk, 0)


@jax.jit
def _run(batch_user, batch_items_flat, rowbase, score_mat):
    mesh = plsc.VectorSubcoreMesh(core_axis_name="c", subcore_axis_name="s")
    f = pl.kernel(
        _body,
        out_type=jax.ShapeDtypeStruct((BATCH * HIST,), jnp.float32),
        mesh=mesh,
        compiler_params=pltpu.CompilerParams(use_tc_tiling_on_sc=True,
                                             needs_layout_passes=False),
        scratch_types=[
            pltpu.VMEM((C,), jnp.int32),
            pltpu.VMEM((C * HIST,), jnp.int32),
            pltpu.VMEM((C * HIST,), jnp.int32),
            pltpu.VMEM((C, NUM_ITEMS), jnp.float32),
            pltpu.VMEM((C * HIST,), jnp.float32),
            pltpu.SemaphoreType.DMA,
        ],
    )
    return f(batch_user, batch_items_flat, rowbase, score_mat)


def kernel(batch_user, batch_items, score_mat):
    rowbase = jnp.arange(C * HIST, dtype=jnp.int32) // HIST
    out_flat = _run(batch_user, batch_items.reshape(-1), rowbase, score_mat)
    return out_flat.reshape(BATCH, HIST)
